# baseline (speedup 1.0000x reference)
import jax
import jax.numpy as jnp
from jax import lax
from jax.experimental import pallas as pl
from jax.experimental.pallas import tpu as pltpu

BT, SQ, NF = 1024, 50, 12
WIDTH = 782
BB = 64
NBLK = BT // BB

SEG_OFF = (0, 32, 59, 187, 315, 342, 369, 396, 524, 652)
COST_COL = (780, 781)


def _block(feat_ref, out_ref):
    bounds = list(SEG_OFF[1:]) + [COST_COL[0], COST_COL[1]]
    col = lax.broadcasted_iota(jnp.int32, (1, 1, WIDTH), 2)
    field = sum((col >= b).astype(jnp.int32) for b in bounds)
    base = jnp.zeros((1, 1, WIDTH), jnp.int32)
    for off in bounds:
        base = jnp.where(col >= off, off, base)
    relf = (col - base).astype(jnp.float32)
    is_cost = col >= COST_COL[0]

    krow = lax.broadcasted_iota(jnp.int32, (NF, WIDTH), 0)
    m = (krow == field.reshape(1, WIDTH)).astype(jnp.bfloat16)
    fb = lax.dot_general(
        feat_ref[...].astype(jnp.bfloat16), m, (((2,), (0,)), ((), ())),
        preferred_element_type=jnp.float32,
    )
    onehot = jnp.where(fb == relf, jnp.float32(1.0), jnp.float32(0.0))
    out_ref[...] = jnp.where(is_cost, fb, onehot)


@jax.jit
def kernel(feature):
    return pl.pallas_call(
        _block,
        grid=(NBLK,),
        in_specs=[pl.BlockSpec((BB, SQ, NF), lambda i: (i, 0, 0))],
        out_specs=pl.BlockSpec((BB, SQ, WIDTH), lambda i: (i, 0, 0)),
        out_shape=jax.ShapeDtypeStruct((BT, SQ, WIDTH), jnp.float32),
        compiler_params=pltpu.CompilerParams(
            dimension_semantics=("parallel",),
        ),
    )(feature)


# speedup vs baseline: 1.0066x; 1.0066x over previous
import jax
import jax.numpy as jnp
from jax import lax
from jax.experimental import pallas as pl
from jax.experimental.pallas import tpu as pltpu

BT, SQ, NF = 1024, 50, 12
WIDTH = 782
BB = 64
NBLK = BT // BB

SEG_OFF = (0, 32, 59, 187, 315, 342, 369, 396, 524, 652)
COST_COL = (780, 781)


def _block(feat_ref, out_ref):
    bounds = list(SEG_OFF[1:]) + [COST_COL[0], COST_COL[1]]
    col = lax.broadcasted_iota(jnp.int32, (1, 1, WIDTH), 2)
    field = sum((col >= b).astype(jnp.int32) for b in bounds)
    base = jnp.zeros((1, 1, WIDTH), jnp.int32)
    for off in bounds:
        base = jnp.where(col >= off, off, base)
    relf = (col - base).astype(jnp.float32)
    is_cost = col >= COST_COL[0]

    krow = lax.broadcasted_iota(jnp.int32, (NF, WIDTH), 0)
    m = (krow == field.reshape(1, WIDTH)).astype(jnp.float32)
    fb = lax.dot_general(
        feat_ref[...], m, (((2,), (0,)), ((), ())),
        preferred_element_type=jnp.float32,
    )
    onehot = jnp.where(fb == relf, jnp.float32(1.0), jnp.float32(0.0))
    out_ref[...] = jnp.where(is_cost, fb, onehot)


@jax.jit
def kernel(feature):
    return pl.pallas_call(
        _block,
        grid=(NBLK,),
        in_specs=[pl.BlockSpec((BB, SQ, NF), lambda i: (i, 0, 0))],
        out_specs=pl.BlockSpec((BB, SQ, WIDTH), lambda i: (i, 0, 0)),
        out_shape=jax.ShapeDtypeStruct((BT, SQ, WIDTH), jnp.float32),
        compiler_params=pltpu.CompilerParams(
            dimension_semantics=("parallel",),
        ),
    )(feature)
